# baseline (speedup 1.0000x reference)
import functools

import jax
import jax.numpy as jnp
from jax import lax
from jax.experimental import pallas as pl
from jax.experimental.pallas import tpu as pltpu, tpu_sc as plsc


_CHUNK = 128


def _sc_gather_all(tail_idx, head_idx, rel_idx, entity_embedding,
                   relation_embedding, origin_relation_embedding):
    N = tail_idx.shape[0]
    B = head_idx.shape[0]
    D = entity_embedding.shape[1]

    info = plsc.get_sparse_core_info()
    NC, NS = info.num_cores, info.num_subcores
    NW = NC * NS
    n_per_w = N // NW
    b_per_w = B // NW
    n_full = n_per_w // _CHUNK
    rem = n_per_w - n_full * _CHUNK

    mesh = plsc.VectorSubcoreMesh(core_axis_name="c", subcore_axis_name="s")

    @functools.partial(
        pl.kernel,
        mesh=mesh,
        out_type=(
            jax.ShapeDtypeStruct((N, D), jnp.float32),
            jax.ShapeDtypeStruct((B, D), jnp.float32),
            jax.ShapeDtypeStruct((B, D), jnp.float32),
            jax.ShapeDtypeStruct((B, D), jnp.float32),
        ),
        scratch_types=[
            pltpu.VMEM((n_per_w,), jnp.int32),
            pltpu.VMEM((_CHUNK, D), jnp.float32),
            pltpu.VMEM((b_per_w,), jnp.int32),
            pltpu.VMEM((b_per_w,), jnp.int32),
            pltpu.VMEM((b_per_w, D), jnp.float32),
            pltpu.SemaphoreType.DMA,
        ],
    )
    def k(tail_idx_hbm, head_idx_hbm, rel_idx_hbm, ent_hbm, rel_hbm, orig_hbm,
          tail_out, head_out, rel_out, orig_out,
          tidx_v, trows_v, hidx_v, ridx_v, srows_v, sem):
        wid = lax.axis_index("s") * NC + lax.axis_index("c")
        base = wid * n_per_w
        sbase = wid * b_per_w

        pltpu.sync_copy(tail_idx_hbm.at[pl.ds(base, n_per_w)], tidx_v)
        pltpu.sync_copy(head_idx_hbm.at[pl.ds(sbase, b_per_w)], hidx_v)
        pltpu.sync_copy(rel_idx_hbm.at[pl.ds(sbase, b_per_w)], ridx_v)

        pltpu.async_copy(ent_hbm.at[hidx_v], srows_v, sem).wait()
        pltpu.sync_copy(srows_v, head_out.at[pl.ds(sbase, b_per_w)])
        pltpu.async_copy(rel_hbm.at[ridx_v], srows_v, sem).wait()
        pltpu.sync_copy(srows_v, rel_out.at[pl.ds(sbase, b_per_w)])
        pltpu.async_copy(orig_hbm.at[ridx_v], srows_v, sem).wait()
        pltpu.sync_copy(srows_v, orig_out.at[pl.ds(sbase, b_per_w)])

        def body(c, _):
            off = c * _CHUNK
            pltpu.async_copy(
                ent_hbm.at[tidx_v.at[pl.ds(off, _CHUNK)]], trows_v, sem
            ).wait()
            pltpu.sync_copy(trows_v, tail_out.at[pl.ds(base + off, _CHUNK)])
            return 0

        lax.fori_loop(0, n_full, body, 0)

        if rem:
            off = n_full * _CHUNK
            pltpu.async_copy(
                ent_hbm.at[tidx_v.at[pl.ds(off, rem)]],
                trows_v.at[pl.ds(0, rem)], sem
            ).wait()
            pltpu.sync_copy(trows_v.at[pl.ds(0, rem)],
                            tail_out.at[pl.ds(base + off, rem)])

    return k(tail_idx, head_idx, rel_idx, entity_embedding,
             relation_embedding, origin_relation_embedding)


def kernel(positive, negative, entity_embedding, relation_embedding,
           origin_relation_embedding):
    B, K = negative.shape[0], negative.shape[1] + 1
    D = entity_embedding.shape[1]
    tail_idx = jnp.concatenate([positive[:, 2:3], negative], axis=1).reshape(-1)
    head_idx = positive[:, 0]
    rel_idx = positive[:, 1]
    tail, head, rel, orig = _sc_gather_all(
        tail_idx, head_idx, rel_idx, entity_embedding, relation_embedding,
        origin_relation_embedding)
    return (head[:, None, :], rel[:, None, :], tail.reshape(B, K, D),
            orig[:, None, :])


# speedup vs baseline: 1.0538x; 1.0538x over previous
import functools

import jax
import jax.numpy as jnp
from jax import lax
from jax.experimental import pallas as pl
from jax.experimental.pallas import tpu as pltpu, tpu_sc as plsc


_CHUNK = 128
_NBUF = 4


def _sc_gather_all(tail_idx, head_idx, rel_idx, entity_embedding,
                   relation_embedding, origin_relation_embedding):
    N = tail_idx.shape[0]
    B = head_idx.shape[0]
    D = entity_embedding.shape[1]

    info = plsc.get_sparse_core_info()
    NC, NS = info.num_cores, info.num_subcores
    NW = NC * NS
    n_per_w = N // NW
    b_per_w = B // NW
    n_full = n_per_w // _CHUNK
    rem = n_per_w - n_full * _CHUNK

    mesh = plsc.VectorSubcoreMesh(core_axis_name="c", subcore_axis_name="s")

    @functools.partial(
        pl.kernel,
        mesh=mesh,
        out_type=(
            jax.ShapeDtypeStruct((N, D), jnp.float32),
            jax.ShapeDtypeStruct((B, D), jnp.float32),
            jax.ShapeDtypeStruct((B, D), jnp.float32),
            jax.ShapeDtypeStruct((B, D), jnp.float32),
        ),
        scratch_types=[
            pltpu.VMEM((n_per_w,), jnp.int32),
            pltpu.VMEM((_NBUF, _CHUNK, D), jnp.float32),
            pltpu.VMEM((b_per_w,), jnp.int32),
            pltpu.VMEM((b_per_w,), jnp.int32),
            pltpu.VMEM((b_per_w, D), jnp.float32),
            pltpu.SemaphoreType.DMA,
        ] + [pltpu.SemaphoreType.DMA] * (2 * _NBUF),
    )
    def k(tail_idx_hbm, head_idx_hbm, rel_idx_hbm, ent_hbm, rel_hbm, orig_hbm,
          tail_out, head_out, rel_out, orig_out,
          tidx_v, trows_v, hidx_v, ridx_v, srows_v, sem, *bufsems):
        gsem = bufsems[:_NBUF]
        wsem = bufsems[_NBUF:]
        wid = lax.axis_index("s") * NC + lax.axis_index("c")
        base = wid * n_per_w
        sbase = wid * b_per_w

        pltpu.sync_copy(tail_idx_hbm.at[pl.ds(base, n_per_w)], tidx_v)
        pltpu.sync_copy(head_idx_hbm.at[pl.ds(sbase, b_per_w)], hidx_v)
        pltpu.sync_copy(rel_idx_hbm.at[pl.ds(sbase, b_per_w)], ridx_v)

        def start_gather(c, j):
            pltpu.make_async_copy(
                ent_hbm.at[tidx_v.at[pl.ds(c * _CHUNK, _CHUNK)]],
                trows_v.at[j], gsem[0]).start()

        def wait_gather(j):
            pltpu.make_async_copy(
                ent_hbm.at[tidx_v.at[pl.ds(0, _CHUNK)]], trows_v.at[j],
                gsem[0]).wait()

        def start_write(c, j):
            pltpu.make_async_copy(
                trows_v.at[j], tail_out.at[pl.ds(base + c * _CHUNK, _CHUNK)],
                wsem[0]).start()

        def wait_write(j):
            pltpu.make_async_copy(
                ent_hbm.at[pl.ds(0, _CHUNK)],
                tail_out.at[pl.ds(base, _CHUNK)], wsem[0]).wait()

        pltpu.async_copy(ent_hbm.at[hidx_v], srows_v, sem).wait()
        pltpu.sync_copy(srows_v, head_out.at[pl.ds(sbase, b_per_w)])
        pltpu.async_copy(rel_hbm.at[ridx_v], srows_v, sem).wait()
        pltpu.sync_copy(srows_v, rel_out.at[pl.ds(sbase, b_per_w)])
        pltpu.async_copy(orig_hbm.at[ridx_v], srows_v, sem).wait()
        pltpu.sync_copy(srows_v, orig_out.at[pl.ds(sbase, b_per_w)])

        def step(c, p, first, last):
            wait_gather(p)
            if not first:
                wait_write(1 - p)
            if not last:
                start_gather(c + 1, 1 - p)
            start_write(c, p)

        start_gather(0, 0)
        step(0, 0, True, False)

        def body(g, _):
            step(2 * g + 1, 1, False, False)
            step(2 * g + 2, 0, False, False)
            return 0

        lax.fori_loop(0, (n_full - 2) // 2, body, 0)

        step(n_full - 1, 1, False, True)
        wait_write(1)

        if rem:
            off = n_full * _CHUNK
            pltpu.async_copy(
                ent_hbm.at[tidx_v.at[pl.ds(off, rem)]],
                trows_v.at[0].at[pl.ds(0, rem)], sem
            ).wait()
            pltpu.sync_copy(trows_v.at[0].at[pl.ds(0, rem)],
                            tail_out.at[pl.ds(base + off, rem)])

    return k(tail_idx, head_idx, rel_idx, entity_embedding,
             relation_embedding, origin_relation_embedding)


def kernel(positive, negative, entity_embedding, relation_embedding,
           origin_relation_embedding):
    B, K = negative.shape[0], negative.shape[1] + 1
    D = entity_embedding.shape[1]
    tail_idx = jnp.concatenate([positive[:, 2:3], negative], axis=1).reshape(-1)
    head_idx = positive[:, 0]
    rel_idx = positive[:, 1]
    tail, head, rel, orig = _sc_gather_all(
        tail_idx, head_idx, rel_idx, entity_embedding, relation_embedding,
        origin_relation_embedding)
    return (head[:, None, :], rel[:, None, :], tail.reshape(B, K, D),
            orig[:, None, :])


# speedup vs baseline: 1.1292x; 1.0716x over previous
import functools

import jax
import jax.numpy as jnp
from jax import lax
from jax.experimental import pallas as pl
from jax.experimental.pallas import tpu as pltpu, tpu_sc as plsc


_CHUNK = 256
_NBUF = 2


def _sc_gather_all(tail_idx, head_idx, rel_idx, entity_embedding,
                   relation_embedding, origin_relation_embedding):
    N = tail_idx.shape[0]
    B = head_idx.shape[0]
    D = entity_embedding.shape[1]

    info = plsc.get_sparse_core_info()
    NC, NS = info.num_cores, info.num_subcores
    NW = NC * NS
    n_per_w = N // NW
    b_per_w = B // NW
    n_full = n_per_w // _CHUNK
    rem = n_per_w - n_full * _CHUNK

    mesh = plsc.VectorSubcoreMesh(core_axis_name="c", subcore_axis_name="s")

    @functools.partial(
        pl.kernel,
        mesh=mesh,
        out_type=(
            jax.ShapeDtypeStruct((N, D), jnp.float32),
            jax.ShapeDtypeStruct((B, D), jnp.float32),
            jax.ShapeDtypeStruct((B, D), jnp.float32),
            jax.ShapeDtypeStruct((B, D), jnp.float32),
        ),
        scratch_types=[
            pltpu.VMEM((n_per_w,), jnp.int32),
            pltpu.VMEM((_NBUF, _CHUNK, D), jnp.float32),
            pltpu.VMEM((b_per_w,), jnp.int32),
            pltpu.VMEM((b_per_w,), jnp.int32),
            pltpu.VMEM((b_per_w, D), jnp.float32),
            pltpu.SemaphoreType.DMA,
        ] + [pltpu.SemaphoreType.DMA] * (2 * _NBUF),
    )
    def k(tail_idx_hbm, head_idx_hbm, rel_idx_hbm, ent_hbm, rel_hbm, orig_hbm,
          tail_out, head_out, rel_out, orig_out,
          tidx_v, trows_v, hidx_v, ridx_v, srows_v, sem, *bufsems):
        gsem = bufsems[:_NBUF]
        wsem = bufsems[_NBUF:]
        wid = lax.axis_index("s") * NC + lax.axis_index("c")
        base = wid * n_per_w
        sbase = wid * b_per_w

        pltpu.sync_copy(tail_idx_hbm.at[pl.ds(base, n_per_w)], tidx_v)
        pltpu.sync_copy(head_idx_hbm.at[pl.ds(sbase, b_per_w)], hidx_v)
        pltpu.sync_copy(rel_idx_hbm.at[pl.ds(sbase, b_per_w)], ridx_v)

        def start_gather(c, j):
            pltpu.make_async_copy(
                ent_hbm.at[tidx_v.at[pl.ds(c * _CHUNK, _CHUNK)]],
                trows_v.at[j], gsem[0]).start()

        def wait_gather(j):
            pltpu.make_async_copy(
                ent_hbm.at[tidx_v.at[pl.ds(0, _CHUNK)]], trows_v.at[j],
                gsem[0]).wait()

        def start_write(c, j):
            pltpu.make_async_copy(
                trows_v.at[j], tail_out.at[pl.ds(base + c * _CHUNK, _CHUNK)],
                wsem[0]).start()

        def wait_write(j):
            pltpu.make_async_copy(
                ent_hbm.at[pl.ds(0, _CHUNK)],
                tail_out.at[pl.ds(base, _CHUNK)], wsem[0]).wait()

        pltpu.async_copy(ent_hbm.at[hidx_v], srows_v, sem).wait()
        pltpu.sync_copy(srows_v, head_out.at[pl.ds(sbase, b_per_w)])
        pltpu.async_copy(rel_hbm.at[ridx_v], srows_v, sem).wait()
        pltpu.sync_copy(srows_v, rel_out.at[pl.ds(sbase, b_per_w)])
        pltpu.async_copy(orig_hbm.at[ridx_v], srows_v, sem).wait()
        pltpu.sync_copy(srows_v, orig_out.at[pl.ds(sbase, b_per_w)])

        def step(c, p, first, last):
            wait_gather(p)
            if not first:
                wait_write(1 - p)
            if not last:
                start_gather(c + 1, 1 - p)
            start_write(c, p)

        start_gather(0, 0)
        step(0, 0, True, False)

        def body(g, _):
            step(2 * g + 1, 1, False, False)
            step(2 * g + 2, 0, False, False)
            return 0

        lax.fori_loop(0, (n_full - 2) // 2, body, 0)

        step(n_full - 1, 1, False, True)
        wait_write(1)

        if rem:
            off = n_full * _CHUNK
            pltpu.async_copy(
                ent_hbm.at[tidx_v.at[pl.ds(off, rem)]],
                trows_v.at[0].at[pl.ds(0, rem)], sem
            ).wait()
            pltpu.sync_copy(trows_v.at[0].at[pl.ds(0, rem)],
                            tail_out.at[pl.ds(base + off, rem)])

    return k(tail_idx, head_idx, rel_idx, entity_embedding,
             relation_embedding, origin_relation_embedding)


def kernel(positive, negative, entity_embedding, relation_embedding,
           origin_relation_embedding):
    B, K = negative.shape[0], negative.shape[1] + 1
    D = entity_embedding.shape[1]
    tail_idx = jnp.concatenate([positive[:, 2:3], negative], axis=1).reshape(-1)
    head_idx = positive[:, 0]
    rel_idx = positive[:, 1]
    tail, head, rel, orig = _sc_gather_all(
        tail_idx, head_idx, rel_idx, entity_embedding, relation_embedding,
        origin_relation_embedding)
    return (head[:, None, :], rel[:, None, :], tail.reshape(B, K, D),
            orig[:, None, :])


# speedup vs baseline: 1.8667x; 1.6531x over previous
import functools

import jax
import jax.numpy as jnp
from jax import lax
from jax.experimental import pallas as pl
from jax.experimental.pallas import tpu as pltpu, tpu_sc as plsc


def _sc_gather_all(tail_idx, last_idx, head_idx, rel_idx, entity_embedding,
                   relation_embedding, origin_relation_embedding):
    B = head_idx.shape[0]
    KA = tail_idx.shape[0] // B
    K = KA + 1
    D = entity_embedding.shape[1]

    info = plsc.get_sparse_core_info()
    NC, NS = info.num_cores, info.num_subcores
    NW = NC * NS
    b_per_w = B // NW

    mesh = plsc.VectorSubcoreMesh(core_axis_name="c", subcore_axis_name="s")

    @functools.partial(
        pl.kernel,
        mesh=mesh,
        out_type=(
            jax.ShapeDtypeStruct((B, K, D), jnp.float32),
            jax.ShapeDtypeStruct((B, 1, D), jnp.float32),
            jax.ShapeDtypeStruct((B, 1, D), jnp.float32),
            jax.ShapeDtypeStruct((B, 1, D), jnp.float32),
        ),
        scratch_types=[
            pltpu.VMEM((b_per_w * KA,), jnp.int32),
            pltpu.VMEM((2, KA, D), jnp.float32),
            pltpu.VMEM((b_per_w,), jnp.int32),
            pltpu.VMEM((b_per_w, 1, D), jnp.float32),
            pltpu.VMEM((b_per_w,), jnp.int32),
            pltpu.VMEM((b_per_w,), jnp.int32),
            pltpu.VMEM((b_per_w, D), jnp.float32),
            pltpu.SemaphoreType.DMA,
            pltpu.SemaphoreType.DMA,
            pltpu.SemaphoreType.DMA,
        ],
    )
    def k(tail_idx_hbm, last_idx_hbm, head_idx_hbm, rel_idx_hbm, ent_hbm,
          rel_hbm, orig_hbm,
          tail_out, head_out, rel_out, orig_out,
          tidx_v, trows_v, lidx_v, lrows_v, hidx_v, ridx_v, srows_v,
          sem, gsem, wsem):
        wid = lax.axis_index("s") * NC + lax.axis_index("c")
        sbase = wid * b_per_w

        pltpu.sync_copy(tail_idx_hbm.at[pl.ds(sbase * KA, b_per_w * KA)], tidx_v)
        pltpu.sync_copy(last_idx_hbm.at[pl.ds(sbase, b_per_w)], lidx_v)
        pltpu.sync_copy(head_idx_hbm.at[pl.ds(sbase, b_per_w)], hidx_v)
        pltpu.sync_copy(rel_idx_hbm.at[pl.ds(sbase, b_per_w)], ridx_v)

        def start_gather(c, p):
            pltpu.make_async_copy(
                ent_hbm.at[tidx_v.at[pl.ds(c * KA, KA)]], trows_v.at[p],
                gsem).start()

        def wait_gather(p):
            pltpu.make_async_copy(
                ent_hbm.at[tidx_v.at[pl.ds(0, KA)]], trows_v.at[p],
                gsem).wait()

        def start_write(c, p):
            pltpu.make_async_copy(
                trows_v.at[p], tail_out.at[sbase + c].at[pl.ds(0, KA)],
                wsem).start()

        def wait_write():
            pltpu.make_async_copy(
                ent_hbm.at[pl.ds(0, KA)],
                tail_out.at[sbase].at[pl.ds(0, KA)], wsem).wait()

        start_gather(0, 0)
        pltpu.async_copy(ent_hbm.at[hidx_v], srows_v, sem).wait()
        pltpu.sync_copy(srows_v, head_out.at[pl.ds(sbase, b_per_w), 0])
        pltpu.async_copy(rel_hbm.at[ridx_v], srows_v, sem).wait()
        pltpu.sync_copy(srows_v, rel_out.at[pl.ds(sbase, b_per_w), 0])
        pltpu.async_copy(orig_hbm.at[ridx_v], srows_v, sem).wait()
        pltpu.sync_copy(srows_v, orig_out.at[pl.ds(sbase, b_per_w), 0])
        pltpu.async_copy(ent_hbm.at[lidx_v], lrows_v.at[:, 0], sem).wait()
        pltpu.sync_copy(
            lrows_v, tail_out.at[pl.ds(sbase, b_per_w), pl.ds(KA, 1)])

        def step(c, p, first, last):
            wait_gather(p)
            if not first:
                wait_write()
            if not last:
                start_gather(c + 1, 1 - p)
            start_write(c, p)

        step(0, 0, True, False)

        def body(g, _):
            step(2 * g + 1, 1, False, False)
            step(2 * g + 2, 0, False, False)
            return 0

        lax.fori_loop(0, (b_per_w - 2) // 2, body, 0)

        step(b_per_w - 1, 1, False, True)
        wait_write()

    return k(tail_idx, last_idx, head_idx, rel_idx, entity_embedding,
             relation_embedding, origin_relation_embedding)


def kernel(positive, negative, entity_embedding, relation_embedding,
           origin_relation_embedding):
    tail_idx = jnp.concatenate(
        [positive[:, 2:3], negative[:, :-1]], axis=1).reshape(-1)
    last_idx = negative[:, -1]
    head_idx = positive[:, 0]
    rel_idx = positive[:, 1]
    tail, head, rel, orig = _sc_gather_all(
        tail_idx, last_idx, head_idx, rel_idx, entity_embedding,
        relation_embedding, origin_relation_embedding)
    return (head, rel, tail, orig)


# speedup vs baseline: 3.2229x; 1.7265x over previous
import functools

import jax
import jax.numpy as jnp
from jax import lax
from jax.experimental import pallas as pl
from jax.experimental.pallas import tpu as pltpu, tpu_sc as plsc

_CHUNK = 256


def _sc_gather_all(tail_idx, head_idx, rel_idx, entity_embedding,
                   relation_embedding, origin_relation_embedding):
    N = tail_idx.shape[0]
    B = head_idx.shape[0]
    D = entity_embedding.shape[1]

    info = plsc.get_sparse_core_info()
    NC, NS = info.num_cores, info.num_subcores
    NW = NC * NS
    n_per_w = N // NW
    b_per_w = B // NW
    n_full = n_per_w // _CHUNK
    rem = n_per_w - n_full * _CHUNK

    mesh = plsc.VectorSubcoreMesh(core_axis_name="c", subcore_axis_name="s")

    @functools.partial(
        pl.kernel,
        mesh=mesh,
        out_type=(
            jax.ShapeDtypeStruct((N, D), jnp.float32),
            jax.ShapeDtypeStruct((B, 1, D), jnp.float32),
            jax.ShapeDtypeStruct((B, 1, D), jnp.float32),
            jax.ShapeDtypeStruct((B, 1, D), jnp.float32),
        ),
        scratch_types=[
            pltpu.VMEM((n_per_w,), jnp.int32),
            pltpu.VMEM((2, _CHUNK, D), jnp.float32),
            pltpu.VMEM((b_per_w,), jnp.int32),
            pltpu.VMEM((b_per_w,), jnp.int32),
            pltpu.VMEM((b_per_w, D), jnp.float32),
            pltpu.SemaphoreType.DMA,
            pltpu.SemaphoreType.DMA,
            pltpu.SemaphoreType.DMA,
        ],
    )
    def k(tail_idx_hbm, head_idx_hbm, rel_idx_hbm, ent_hbm, rel_hbm, orig_hbm,
          tail_out, head_out, rel_out, orig_out,
          tidx_v, trows_v, hidx_v, ridx_v, srows_v, sem, gsem, wsem):
        wid = lax.axis_index("s") * NC + lax.axis_index("c")
        base = wid * n_per_w
        sbase = wid * b_per_w

        pltpu.sync_copy(tail_idx_hbm.at[pl.ds(base, n_per_w)], tidx_v)
        pltpu.sync_copy(head_idx_hbm.at[pl.ds(sbase, b_per_w)], hidx_v)
        pltpu.sync_copy(rel_idx_hbm.at[pl.ds(sbase, b_per_w)], ridx_v)

        def start_gather(c, p):
            pltpu.make_async_copy(
                ent_hbm.at[tidx_v.at[pl.ds(c * _CHUNK, _CHUNK)]],
                trows_v.at[p], gsem).start()

        def wait_gather(p):
            pltpu.make_async_copy(
                ent_hbm.at[tidx_v.at[pl.ds(0, _CHUNK)]], trows_v.at[p],
                gsem).wait()

        def start_write(c, p):
            pltpu.make_async_copy(
                trows_v.at[p], tail_out.at[pl.ds(base + c * _CHUNK, _CHUNK)],
                wsem).start()

        def wait_write():
            pltpu.make_async_copy(
                ent_hbm.at[pl.ds(0, _CHUNK)],
                tail_out.at[pl.ds(base, _CHUNK)], wsem).wait()

        start_gather(0, 0)
        pltpu.async_copy(ent_hbm.at[hidx_v], srows_v, sem).wait()
        pltpu.sync_copy(srows_v, head_out.at[pl.ds(sbase, b_per_w), 0])
        pltpu.async_copy(rel_hbm.at[ridx_v], srows_v, sem).wait()
        pltpu.sync_copy(srows_v, rel_out.at[pl.ds(sbase, b_per_w), 0])
        pltpu.async_copy(orig_hbm.at[ridx_v], srows_v, sem).wait()
        pltpu.sync_copy(srows_v, orig_out.at[pl.ds(sbase, b_per_w), 0])

        def step(c, p, first, last):
            wait_gather(p)
            if not first:
                wait_write()
            if not last:
                start_gather(c + 1, 1 - p)
            start_write(c, p)

        step(0, 0, True, False)

        def body(g, _):
            step(2 * g + 1, 1, False, False)
            step(2 * g + 2, 0, False, False)
            return 0

        lax.fori_loop(0, (n_full - 2) // 2, body, 0)

        step(n_full - 1, 1, False, True)
        wait_write()

        if rem:
            off = n_full * _CHUNK
            pltpu.async_copy(
                ent_hbm.at[tidx_v.at[pl.ds(off, rem)]],
                trows_v.at[0].at[pl.ds(0, rem)], sem
            ).wait()
            pltpu.sync_copy(trows_v.at[0].at[pl.ds(0, rem)],
                            tail_out.at[pl.ds(base + off, rem)])

    return k(tail_idx, head_idx, rel_idx, entity_embedding,
             relation_embedding, origin_relation_embedding)


def kernel(positive, negative, entity_embedding, relation_embedding,
           origin_relation_embedding):
    B, K = negative.shape[0], negative.shape[1] + 1
    D = entity_embedding.shape[1]
    tail_idx = jnp.concatenate(
        [positive[:, 2:3], negative], axis=1).T.reshape(-1)
    head_idx = positive[:, 0]
    rel_idx = positive[:, 1]
    tail, head, rel, orig = _sc_gather_all(
        tail_idx, head_idx, rel_idx, entity_embedding, relation_embedding,
        origin_relation_embedding)
    return (head, rel, tail.reshape(K, B, D).transpose(1, 0, 2), orig)
